# baseline (speedup 1.0000x reference)
import functools

import jax
import jax.numpy as jnp
from jax import lax
from jax.experimental import pallas as pl
from jax.experimental.pallas import tpu as pltpu
from jax.experimental.pallas import tpu_sc as plsc

HIDDEN = 2048
NUM_EXPERTS = 64
TOKENS = 16384
NCHUNK = 2
CHUNK = TOKENS // NCHUNK
BT = 2048

NC = 2
NS = 16
L = 16
NW = NC * NS
TPW = CHUNK // NW
GROUPS = TPW // L


def _logits_block(x_ref, w_ref, out_ref):
    out_ref[...] = jax.lax.dot_general(
        w_ref[...], x_ref[...],
        dimension_numbers=(((1,), (1,)), ((), ())),
        preferred_element_type=jnp.float32,
    )


def _tc_logits(x, weight):
    grid = (CHUNK // BT,)
    return pl.pallas_call(
        _logits_block,
        grid=grid,
        in_specs=[
            pl.BlockSpec((BT, HIDDEN), lambda i: (i, 0)),
            pl.BlockSpec((NUM_EXPERTS, HIDDEN), lambda i: (0, 0)),
        ],
        out_specs=pl.BlockSpec((NUM_EXPERTS, BT), lambda i: (0, i)),
        out_shape=jax.ShapeDtypeStruct((NUM_EXPERTS, CHUNK), jnp.float32),
    )(x, weight)


def _sc_select(logits):
    mesh = plsc.VectorSubcoreMesh(core_axis_name="c", subcore_axis_name="s")

    @functools.partial(
        pl.kernel,
        mesh=mesh,
        out_type=[
            jax.ShapeDtypeStruct((2, CHUNK), jnp.float32),
            jax.ShapeDtypeStruct((2, CHUNK), jnp.int32),
        ],
        scratch_types=[
            pltpu.VMEM((NUM_EXPERTS, TPW), jnp.float32),
            pltpu.VMEM((2, TPW), jnp.float32),
            pltpu.VMEM((2, TPW), jnp.int32),
        ],
    )
    def sc_kernel(logits_hbm, outw_hbm, outi_hbm, lbuf, wbuf, ibuf):
        wid = lax.axis_index("s") * NC + lax.axis_index("c")
        base = wid * TPW
        pltpu.sync_copy(logits_hbm.at[:, pl.ds(base, TPW)], lbuf)

        def group_body(g, carry):
            off = g * L
            neg = jnp.full((L,), -jnp.inf, dtype=jnp.float32)
            m0, m1 = neg, neg
            zero = jnp.zeros((L,), dtype=jnp.int32)
            i0, i1 = zero, zero
            for e in range(NUM_EXPERTS):
                v = lbuf[e, pl.ds(off, L)]
                evec = jnp.full((L,), e, dtype=jnp.int32)
                gt0 = v > m0
                gt1 = v > m1
                m1 = jnp.where(gt0, m0, jnp.where(gt1, v, m1))
                i1 = jnp.where(gt0, i0, jnp.where(gt1, evec, i1))
                m0 = jnp.where(gt0, v, m0)
                i0 = jnp.where(gt0, evec, i0)
            e1 = jnp.exp(m1 - m0)
            denom = 1.0 + e1
            wbuf[0, pl.ds(off, L)] = 1.0 / denom
            wbuf[1, pl.ds(off, L)] = e1 / denom
            ibuf[0, pl.ds(off, L)] = i0
            ibuf[1, pl.ds(off, L)] = i1
            return carry

        lax.fori_loop(0, GROUPS, group_body, 0)
        pltpu.sync_copy(wbuf, outw_hbm.at[:, pl.ds(base, TPW)])
        pltpu.sync_copy(ibuf, outi_hbm.at[:, pl.ds(base, TPW)])

    return sc_kernel(logits)


@jax.jit
def kernel(x, weight):
    outs = []
    for c in range(NCHUNK):
        logits = _tc_logits(lax.slice_in_dim(x, c * CHUNK, (c + 1) * CHUNK),
                            weight)
        outs.append(_sc_select(logits))
    outw = jnp.concatenate([o[0] for o in outs], axis=1)
    outi = jnp.concatenate([o[1] for o in outs], axis=1)
    return (outw.T, outi.T)


# speedup vs baseline: 2.5892x; 2.5892x over previous
import jax
import jax.numpy as jnp
from jax.experimental import pallas as pl

HIDDEN = 2048
NUM_EXPERTS = 64
TOKENS = 16384
BT = 1024
HALF = TOKENS // 2


def _top2_softmax(logits, wout_ref, iout_ref):
    idx = jax.lax.broadcasted_iota(jnp.int32, logits.shape, 1)
    m0 = jnp.max(logits, axis=-1, keepdims=True)
    i0 = jnp.min(jnp.where(logits == m0, idx, NUM_EXPERTS), axis=-1,
                 keepdims=True)
    masked = jnp.where(idx == i0, -jnp.inf, logits)
    m1 = jnp.max(masked, axis=-1, keepdims=True)
    i1 = jnp.min(jnp.where(masked == m1, idx, NUM_EXPERTS), axis=-1,
                 keepdims=True)
    e1 = jnp.exp(m1 - m0)
    denom = 1.0 + e1
    wout_ref[...] = jnp.concatenate([1.0 / denom, e1 / denom], axis=-1)
    iout_ref[...] = jnp.concatenate([i0, i1], axis=-1)


def _router_block(xa_ref, xb_ref, w_ref, wa_ref, ia_ref, wb_ref, ib_ref):
    w = w_ref[...]
    dn = (((1,), (1,)), ((), ()))
    la = jax.lax.dot_general(xa_ref[0], w, dimension_numbers=dn,
                             preferred_element_type=jnp.float32)
    _top2_softmax(la, wa_ref, ia_ref)
    lb = jax.lax.dot_general(xb_ref[0], w, dimension_numbers=dn,
                             preferred_element_type=jnp.float32)
    _top2_softmax(lb, wb_ref, ib_ref)


@jax.jit
def kernel(x, weight):
    grid = (HALF // BT,)
    x3 = x.reshape(2, HALF, HIDDEN)
    wa, ia, wb, ib = pl.pallas_call(
        _router_block,
        grid=grid,
        in_specs=[
            pl.BlockSpec((1, BT, HIDDEN), lambda i: (0, i, 0)),
            pl.BlockSpec((1, BT, HIDDEN), lambda i: (1, i, 0)),
            pl.BlockSpec((NUM_EXPERTS, HIDDEN), lambda i: (0, 0)),
        ],
        out_specs=[
            pl.BlockSpec((BT, 2), lambda i: (i, 0)),
            pl.BlockSpec((BT, 2), lambda i: (i, 0)),
            pl.BlockSpec((BT, 2), lambda i: (i, 0)),
            pl.BlockSpec((BT, 2), lambda i: (i, 0)),
        ],
        out_shape=[
            jax.ShapeDtypeStruct((HALF, 2), jnp.float32),
            jax.ShapeDtypeStruct((HALF, 2), jnp.int32),
            jax.ShapeDtypeStruct((HALF, 2), jnp.float32),
            jax.ShapeDtypeStruct((HALF, 2), jnp.int32),
        ],
    )(x3, x3, weight)
    return (jnp.concatenate([wa, wb], axis=0),
            jnp.concatenate([ia, ib], axis=0))


# speedup vs baseline: 2.6259x; 1.0142x over previous
import jax
import jax.numpy as jnp
from jax.experimental import pallas as pl

HIDDEN = 2048
NUM_EXPERTS = 64
TOKENS = 16384
NSTREAM = 4
BT = 512
PART = TOKENS // NSTREAM


def _top2_softmax(logits, wout_ref, iout_ref):
    idx = jax.lax.broadcasted_iota(jnp.int32, logits.shape, 1)
    m0 = jnp.max(logits, axis=-1, keepdims=True)
    i0 = jnp.min(jnp.where(logits == m0, idx, NUM_EXPERTS), axis=-1,
                 keepdims=True)
    masked = jnp.where(idx == i0, -jnp.inf, logits)
    m1 = jnp.max(masked, axis=-1, keepdims=True)
    i1 = jnp.min(jnp.where(masked == m1, idx, NUM_EXPERTS), axis=-1,
                 keepdims=True)
    e1 = jnp.exp(m1 - m0)
    denom = 1.0 + e1
    wout_ref[...] = jnp.concatenate([1.0 / denom, e1 / denom], axis=-1)
    iout_ref[...] = jnp.concatenate([i0, i1], axis=-1)


def _router_block(*refs):
    x_refs = refs[:NSTREAM]
    w = refs[NSTREAM][...]
    out_refs = refs[NSTREAM + 1:]
    dn = (((1,), (1,)), ((), ()))
    for s in range(NSTREAM):
        logits = jax.lax.dot_general(x_refs[s][0], w, dimension_numbers=dn,
                                     preferred_element_type=jnp.float32)
        _top2_softmax(logits, out_refs[2 * s], out_refs[2 * s + 1])


@jax.jit
def kernel(x, weight):
    grid = (PART // BT,)
    x4 = x.reshape(NSTREAM, PART, HIDDEN)

    def make_xspec(s):
        return pl.BlockSpec((1, BT, HIDDEN), lambda i, s=s: (s, i, 0))

    out_specs, out_shape = [], []
    for _ in range(NSTREAM):
        out_specs += [pl.BlockSpec((BT, 2), lambda i: (i, 0)),
                      pl.BlockSpec((BT, 2), lambda i: (i, 0))]
        out_shape += [jax.ShapeDtypeStruct((PART, 2), jnp.float32),
                      jax.ShapeDtypeStruct((PART, 2), jnp.int32)]

    outs = pl.pallas_call(
        _router_block,
        grid=grid,
        in_specs=[make_xspec(s) for s in range(NSTREAM)]
        + [pl.BlockSpec((NUM_EXPERTS, HIDDEN), lambda i: (0, 0))],
        out_specs=out_specs,
        out_shape=out_shape,
    )(*([x4] * NSTREAM), weight)
    return (jnp.concatenate(outs[0::2], axis=0),
            jnp.concatenate(outs[1::2], axis=0))
